# baseline (speedup 1.0000x reference)
import functools

import jax
import jax.numpy as jnp
from jax.experimental import pallas as pl
from jax.experimental.pallas import tpu as pltpu

STEP_SIZE = 0.1


def _main_kernel(a_top_ref, a_bot_ref, xs_ref, y_ref, d_out_ref, *, bk, n):
    ki = pl.program_id(0)
    nk = pl.num_programs(0)
    base = ki * bk
    tail = n - (nk - 1) * bk
    nh = n // 2

    deg = (jnp.sum(a_top_ref[...], axis=0, keepdims=True)
           + jnp.sum(a_bot_ref[...], axis=0, keepdims=True) + 1.0)
    d_row = 1.0 / jnp.sqrt(deg)
    d_row = jnp.where(jnp.isinf(d_row), 0.0, d_row)

    ri = jax.lax.broadcasted_iota(jnp.int32, (bk, bk), 0)
    ci = jax.lax.broadcasted_iota(jnp.int32, (bk, bk), 1)
    keep = (ri == ci) & (ci + base < n)
    dmat = jnp.where(keep, jnp.broadcast_to(d_row, (bk, bk)), 0.0)
    d_col = jnp.sum(dmat, axis=1, keepdims=True)
    d_out_ref[...] = d_col

    subl = jax.lax.broadcasted_iota(jnp.int32, (bk, 1), 0)
    row_valid = (base + subl) < n

    xs = jnp.where(row_valid, d_col * xs_ref[...], 0.0)

    @pl.when(ki == 0)
    def _init():
        y_ref[...] = jnp.zeros_like(y_ref)

    @pl.when(ki < nk - 1)
    def _body():
        y_ref[pl.ds(0, nh), :] += jnp.dot(
            a_top_ref[...], xs,
            preferred_element_type=jnp.float32).astype(jnp.bfloat16)
        y_ref[pl.ds(nh, nh), :] += jnp.dot(
            a_bot_ref[...], xs,
            preferred_element_type=jnp.float32).astype(jnp.bfloat16)
        y_ref[pl.ds(base, bk), :] += xs.astype(jnp.bfloat16)

    @pl.when(ki == nk - 1)
    def _body_tail():
        lane = jax.lax.broadcasted_iota(jnp.int32, (1, bk), 1)
        valid = (nk - 1) * bk + lane < n
        a_t = jnp.where(valid, a_top_ref[...], 0.0)
        a_b = jnp.where(valid, a_bot_ref[...], 0.0)
        y_ref[pl.ds(0, nh), :] += jnp.dot(
            a_t, xs,
            preferred_element_type=jnp.float32).astype(jnp.bfloat16)
        y_ref[pl.ds(nh, nh), :] += jnp.dot(
            a_b, xs,
            preferred_element_type=jnp.float32).astype(jnp.bfloat16)
        y_ref[pl.ds((nk - 1) * bk, tail), :] += xs[0:tail, :].astype(jnp.bfloat16)


def _epilogue_kernel(y_ref, d_ref, x_ref, w_ref, out_ref):
    y = d_ref[...] * y_ref[...].astype(jnp.float32)
    inter = jnp.dot(y, w_ref[...], preferred_element_type=jnp.float32)
    out_ref[...] = x_ref[...] + STEP_SIZE * inter


def kernel(x, A, W):
    n, d_feat = x.shape
    w_star = 0.5 * (W + W.T)

    bk = 640
    nk = -(-n // bk)

    y, d = pl.pallas_call(
        functools.partial(_main_kernel, bk=bk, n=n),
        grid=(nk,),
        in_specs=[
            pl.BlockSpec((n // 2, bk), lambda k: (0, k)),
            pl.BlockSpec((n // 2, bk), lambda k: (1, k)),
            pl.BlockSpec((bk, d_feat), lambda k: (k, 0)),
        ],
        out_specs=[
            pl.BlockSpec((n, d_feat), lambda k: (0, 0)),
            pl.BlockSpec((bk, 1), lambda k: (k, 0)),
        ],
        out_shape=[
            jax.ShapeDtypeStruct((n, d_feat), jnp.bfloat16),
            jax.ShapeDtypeStruct((n, 1), jnp.float32),
        ],
    )(A, A, x)

    bm = next(b for b in (1000, 500, 250, 200, 100, 50, 40, 25, 20, 10,
                          8, 5, 4, 2, 1) if n % b == 0)
    out = pl.pallas_call(
        _epilogue_kernel,
        grid=(n // bm,),
        in_specs=[
            pl.BlockSpec((bm, d_feat), lambda k: (k, 0)),
            pl.BlockSpec((bm, 1), lambda k: (k, 0)),
            pl.BlockSpec((bm, d_feat), lambda k: (k, 0)),
            pl.BlockSpec((d_feat, d_feat), lambda k: (0, 0)),
        ],
        out_specs=pl.BlockSpec((bm, d_feat), lambda k: (k, 0)),
        out_shape=jax.ShapeDtypeStruct((n, d_feat), jnp.float32),
    )(y, d, x, w_star)
    return out


# speedup vs baseline: 1.0103x; 1.0103x over previous
import functools

import jax
import jax.numpy as jnp
from jax.experimental import pallas as pl
from jax.experimental.pallas import tpu as pltpu

STEP_SIZE = 0.1


def _gcn_kernel(a_top_ref, a_bot_ref, xs_ref, xf_ref, w_ref, out_ref, d_ref,
                *, bk, n):
    ki = pl.program_id(0)
    nk = pl.num_programs(0)
    base = ki * bk
    tail = n - (nk - 1) * bk
    nh = n // 2

    deg = (jnp.sum(a_top_ref[...], axis=0, keepdims=True)
           + jnp.sum(a_bot_ref[...], axis=0, keepdims=True) + 1.0)
    d_row = 1.0 / jnp.sqrt(deg)
    d_row = jnp.where(jnp.isinf(d_row), 0.0, d_row)

    ri = jax.lax.broadcasted_iota(jnp.int32, (bk, bk), 0)
    ci = jax.lax.broadcasted_iota(jnp.int32, (bk, bk), 1)
    keep = (ri == ci) & (ci + base < n)
    dmat = jnp.where(keep, jnp.broadcast_to(d_row, (bk, bk)), 0.0)
    d_col = jnp.sum(dmat, axis=1, keepdims=True)
    d_ref[pl.ds(base, bk), :] = d_col

    subl = jax.lax.broadcasted_iota(jnp.int32, (bk, 1), 0)
    row_valid = (base + subl) < n

    xs = jnp.where(row_valid, d_col * xs_ref[...], 0.0)

    @pl.when(ki == 0)
    def _init():
        out_ref[...] = jnp.zeros_like(out_ref)

    @pl.when(ki < nk - 1)
    def _body():
        out_ref[pl.ds(0, nh), :] += jnp.dot(
            a_top_ref[...], xs, preferred_element_type=jnp.float32)
        out_ref[pl.ds(nh, nh), :] += jnp.dot(
            a_bot_ref[...], xs, preferred_element_type=jnp.float32)
        out_ref[pl.ds(base, bk), :] += xs

    @pl.when(ki == nk - 1)
    def _body_tail():
        lane = jax.lax.broadcasted_iota(jnp.int32, (1, bk), 1)
        valid = (nk - 1) * bk + lane < n
        a_t = jnp.where(valid, a_top_ref[...], 0.0)
        a_b = jnp.where(valid, a_bot_ref[...], 0.0)
        out_ref[pl.ds(0, nh), :] += jnp.dot(
            a_t, xs, preferred_element_type=jnp.float32)
        out_ref[pl.ds(nh, nh), :] += jnp.dot(
            a_b, xs, preferred_element_type=jnp.float32)
        out_ref[pl.ds((nk - 1) * bk, tail), :] += xs[0:tail, :]
        y = d_ref[pl.ds(0, n), :] * out_ref[...]
        inter = jnp.dot(y, w_ref[...], preferred_element_type=jnp.float32)
        out_ref[...] = xf_ref[...] + STEP_SIZE * inter


def kernel(x, A, W):
    n, d_feat = x.shape
    w_star = 0.5 * (W + W.T)

    bk = 384
    nk = -(-n // bk)
    n_pad = nk * bk

    return pl.pallas_call(
        functools.partial(_gcn_kernel, bk=bk, n=n),
        grid=(nk,),
        in_specs=[
            pl.BlockSpec((n // 2, bk), lambda k: (0, k)),
            pl.BlockSpec((n // 2, bk), lambda k: (1, k)),
            pl.BlockSpec((bk, d_feat), lambda k: (k, 0)),
            pl.BlockSpec((n, d_feat), lambda k: (0, 0)),
            pl.BlockSpec((d_feat, d_feat), lambda k: (0, 0)),
        ],
        out_specs=pl.BlockSpec((n, d_feat), lambda k: (0, 0)),
        out_shape=jax.ShapeDtypeStruct((n, d_feat), jnp.float32),
        scratch_shapes=[
            pltpu.VMEM((n_pad, 1), jnp.float32),
        ],
    )(A, A, x, x, w_star)
